# baseline (speedup 1.0000x reference)
import functools

import jax
import jax.numpy as jnp
from jax import lax
from jax.experimental import pallas as pl
from jax.experimental.pallas import tpu as pltpu
from jax.experimental.pallas import tpu_sc as plsc

N_NODES = 10000
N_EDGES = 320000
D_FEAT = 128
D_EDGE = 16
HID = 72
HP = 80
HPE = 128
NPAD = 10240
EPAD = 327680
NW = 32
EPW = EPAD // NW
EB = 128
NBLK = EPW // EB
RPW = NPAD // 16
NBUF = 4
NGRP = NBLK // NBUF


def _mesh():
    return plsc.VectorSubcoreMesh(core_axis_name="c", subcore_axis_name="s")


_SC_PARAMS = pltpu.CompilerParams(use_tc_tiling_on_sc=False)


@functools.lru_cache(maxsize=None)
def _deg_kernel():
    @functools.partial(
        pl.kernel,
        out_type=jax.ShapeDtypeStruct((2, NPAD), jnp.float32),
        mesh=_mesh(),
        compiler_params=_SC_PARAMS,
        scratch_types=[
            pltpu.VMEM((NBLK, EB), jnp.int32),
            pltpu.VMEM((EB,), jnp.float32),
            pltpu.VMEM_SHARED((NPAD,), jnp.float32),
            pltpu.SemaphoreType.DMA,
        ],
    )
    def deg_k(col2_hbm, zero_hbm, out_hbm, cidx2, ones_v, deg_sp, sem):
        c = lax.axis_index("c")
        s = lax.axis_index("s")
        wid = s * 2 + c
        pltpu.sync_copy(zero_hbm.at[pl.ds(s * RPW, RPW)],
                        deg_sp.at[pl.ds(s * RPW, RPW)])
        pltpu.sync_copy(col2_hbm.at[pl.ds(wid * NBLK, NBLK)], cidx2)
        for j in range(EB // 16):
            ones_v[pl.ds(j * 16, 16)] = jnp.ones((16,), jnp.float32)
        plsc.subcore_barrier()

        def grp(G, carry):
            g0 = G * NBUF
            descs = [
                pltpu.async_copy(ones_v, deg_sp.at[cidx2.at[g0 + b]], sem,
                                 add=True)
                for b in range(NBUF)
            ]
            for d in descs:
                d.wait()
            return carry

        lax.fori_loop(0, NGRP, grp, 0)
        plsc.subcore_barrier()
        pltpu.sync_copy(deg_sp.at[pl.ds(s * RPW, RPW)],
                        out_hbm.at[c, pl.ds(s * RPW, RPW)])

    return deg_k


@functools.lru_cache(maxsize=None)
def _agg_kernel():
    @functools.partial(
        pl.kernel,
        out_type=jax.ShapeDtypeStruct((2, NPAD, HP), jnp.float32),
        mesh=_mesh(),
        compiler_params=_SC_PARAMS,
        scratch_types=[
            pltpu.VMEM((NBLK, EB), jnp.int32),
            pltpu.VMEM((NBLK, EB), jnp.int32),
            pltpu.VMEM((NBUF * EB, HP), jnp.float32),
            pltpu.VMEM_SHARED((NPAD, HP), jnp.float32),
            pltpu.SemaphoreType.DMA,
            pltpu.SemaphoreType.DMA,
        ],
    )
    def agg_k(tab_hbm, row2_hbm, col2_hbm, zero_hbm, out_hbm,
              ridx2, cidx2, gball, agg_sp, gsem, ssem):
        c = lax.axis_index("c")
        s = lax.axis_index("s")
        wid = s * 2 + c
        for j in range(RPW // EB):
            pltpu.sync_copy(zero_hbm,
                            agg_sp.at[pl.ds(s * RPW + j * EB, EB)])
        pltpu.sync_copy(row2_hbm.at[pl.ds(wid * NBLK, NBLK)], ridx2)
        pltpu.sync_copy(col2_hbm.at[pl.ds(wid * NBLK, NBLK)], cidx2)
        plsc.subcore_barrier()

        NB2 = NBUF // 2
        NG2 = NBLK // NB2

        def gath(G, half):
            return [
                pltpu.async_copy(
                    tab_hbm.at[ridx2.at[G * NB2 + b]],
                    gball.at[pl.ds((half * NB2 + b) * EB, EB)], gsem)
                for b in range(NB2)
            ]

        def scat(G, half):
            return [
                pltpu.async_copy(
                    gball.at[pl.ds((half * NB2 + b) * EB, EB)],
                    agg_sp.at[cidx2.at[G * NB2 + b]], ssem, add=True)
                for b in range(NB2)
            ]

        gds = gath(0, 0)
        sds_prev = None
        sds_tail = []
        for G in range(NG2):
            h = G % 2
            for d in gds:
                d.wait()
            sds = scat(G, h)
            if G + 1 < NG2:
                if sds_prev is not None:
                    for d in sds_prev:
                        d.wait()
                gds = gath(G + 1, 1 - h)
            else:
                sds_tail = sds_prev or []
            sds_prev = sds
        for d in sds_tail + sds_prev:
            d.wait()
        plsc.subcore_barrier()
        pltpu.sync_copy(agg_sp.at[pl.ds(s * RPW, RPW)],
                        out_hbm.at[c, pl.ds(s * RPW, RPW)])

    return agg_k


@functools.lru_cache(maxsize=None)
def _edge_kernel():
    @functools.partial(
        pl.kernel,
        out_type=jax.ShapeDtypeStruct((EPAD, HPE), jnp.bfloat16),
        mesh=_mesh(),
        compiler_params=_SC_PARAMS,
        scratch_types=[
            pltpu.VMEM((NBLK, EB), jnp.int32),
            pltpu.VMEM((NBLK, EB), jnp.int32),
            pltpu.VMEM((4, EB), jnp.int32),
            pltpu.VMEM((4 * EB, HPE), jnp.bfloat16),
            pltpu.VMEM((4 * EB, HPE), jnp.bfloat16),
            pltpu.VMEM_SHARED((16 * 4 * EB, HPE), jnp.bfloat16),
            pltpu.SemaphoreType.DMA,
            pltpu.SemaphoreType.DMA,
            pltpu.SemaphoreType.DMA,
        ],
    )
    def edge_k(u_hbm, w_hbm, row2_hbm, col2_hbm, id4_hbm, out_hbm,
               ridx2, cidx2, iref, uball, wball, zone, gsem, asem, wsem):
        c = lax.axis_index("c")
        s = lax.axis_index("s")
        wid = s * 2 + c
        pltpu.sync_copy(row2_hbm.at[pl.ds(wid * NBLK, NBLK)], ridx2)
        pltpu.sync_copy(col2_hbm.at[pl.ds(wid * NBLK, NBLK)], cidx2)
        pltpu.sync_copy(id4_hbm, iref)
        for r in range(4):
            for j in range(EB // 16):
                sl = pl.ds(j * 16, 16)
                iref[r, sl] = iref[r, sl] + s * (4 * EB)
        base = wid * EPW

        NB2 = 2
        NG2 = NBLK // NB2

        def zsl(half, b):
            return pl.ds((s * 4 + half * NB2 + b) * EB, EB)

        def gath(G, half):
            gds = []
            for b in range(NB2):
                sl = pl.ds((half * NB2 + b) * EB, EB)
                gds.append(pltpu.async_copy(
                    u_hbm.at[ridx2.at[G * NB2 + b]], uball.at[sl], gsem))
                gds.append(pltpu.async_copy(
                    w_hbm.at[cidx2.at[G * NB2 + b]], wball.at[sl], gsem))
            return gds

        def setu(half):
            return [
                pltpu.async_copy(
                    uball.at[pl.ds((half * NB2 + b) * EB, EB)],
                    zone.at[zsl(half, b)], asem)
                for b in range(NB2)
            ]

        def addw(half):
            return [
                pltpu.async_copy(
                    wball.at[pl.ds((half * NB2 + b) * EB, EB)],
                    zone.at[iref.at[half * NB2 + b]], asem, add=True)
                for b in range(NB2)
            ]

        def wrt(G, half):
            return [
                pltpu.async_copy(
                    zone.at[zsl(half, b)],
                    out_hbm.at[pl.ds(base + (G * NB2 + b) * EB, EB)], wsem)
                for b in range(NB2)
            ]

        gds = gath(0, 0)
        wds_prev = None
        wds_tail = []
        for G in range(NG2):
            h = G % 2
            for d in gds:
                d.wait()
            for d in setu(h):
                d.wait()
            for d in addw(h):
                d.wait()
            wds = wrt(G, h)
            if G + 1 < NG2:
                if wds_prev is not None:
                    for d in wds_prev:
                        d.wait()
                gds = gath(G + 1, 1 - h)
            else:
                wds_tail = wds_prev or []
            wds_prev = wds
        for d in wds_tail + wds_prev:
            d.wait()

    return edge_k


def _dis_from_degT(degT):
    deg = degT[:, 0:1] + degT[:, 1:2]
    return jnp.where(deg > 0.0, lax.rsqrt(jnp.maximum(deg, 1e-12)), 0.0)


def _tc_pre_body(x_ref, nw_ref, nb_ref, degT_ref, iw_ref, rw_ref, cb_ref,
                 hw_ref, hr_ref):
    dis = _dis_from_degT(degT_ref[...])
    h0 = jnp.maximum(
        jnp.dot(x_ref[...], nw_ref[...], preferred_element_type=jnp.float32)
        + nb_ref[...], 0.0)
    hw_ref[...] = dis * jnp.dot(h0, iw_ref[...],
                                preferred_element_type=jnp.float32)
    hr_ref[...] = jnp.dot(h0, rw_ref[...],
                          preferred_element_type=jnp.float32) + cb_ref[...]


def _bn_layer(aggp, degT, hr, g, b):
    dis = _dis_from_degT(degT)
    agg = (aggp[0] + aggp[1])[:, :HID] * dis
    h2 = jnp.maximum(agg + hr, 0.0)
    mask = (lax.broadcasted_iota(jnp.int32, (NPAD, 1), 0)
            < N_NODES).astype(jnp.float32)
    h2m = h2 * mask
    mu = jnp.sum(h2m, axis=0, keepdims=True) / N_NODES
    var = jnp.sum((h2 - mu) ** 2 * mask, axis=0, keepdims=True) / N_NODES
    hn = (h2 - mu) * lax.rsqrt(var + 1e-5) * g + b
    return hn, dis


def _tc_mid_body(aggp_ref, degT_ref, hr_ref, g_ref, b_ref,
                 iw_ref, rw_ref, cb_ref, hw_ref, hrn_ref):
    hn, dis = _bn_layer(aggp_ref[...], degT_ref[...], hr_ref[...],
                        g_ref[...], b_ref[...])
    hw_ref[...] = dis * jnp.dot(hn, iw_ref[...],
                                preferred_element_type=jnp.float32)
    hrn_ref[...] = jnp.dot(hn, rw_ref[...],
                           preferred_element_type=jnp.float32) + cb_ref[...]


def _tc_post_body(aggp_ref, degT_ref, hr_ref, g_ref, b_ref,
                  e1w_ref, e1b_ref, e2w_ref, e2b_ref,
                  m1t_ref, m1bh_ref, m1b_ref,
                  u_ref, w_ref, a3_ref):
    hn, _ = _bn_layer(aggp_ref[...], degT_ref[...], hr_ref[...],
                      g_ref[...], b_ref[...])
    e1t = e1w_ref[0:HID, :]
    e1bm = e1w_ref[HID:2 * HID, :]
    m1t = m1t_ref[...]
    m1bh = m1bh_ref[...]
    bias = (jnp.dot(e1b_ref[...], m1t, preferred_element_type=jnp.float32)
            + jnp.dot(e2b_ref[...], m1bh, preferred_element_type=jnp.float32)
            + m1b_ref[...])
    u_ref[...] = (jnp.dot(
        jnp.dot(hn, e1t, preferred_element_type=jnp.float32), m1t,
        preferred_element_type=jnp.float32) + bias).astype(jnp.bfloat16)
    w_ref[...] = jnp.dot(
        jnp.dot(hn, e1bm, preferred_element_type=jnp.float32), m1t,
        preferred_element_type=jnp.float32).astype(jnp.bfloat16)
    a3_ref[...] = jnp.dot(e2w_ref[...], m1bh,
                          preferred_element_type=jnp.float32)


EBLK = 2560


def _tc_edge_body(g_ref, attr_ref, a3_ref, m2w_ref, m2b_ref, out_ref):
    g = g_ref[...].reshape(EBLK, HPE)
    p = g + jnp.dot(attr_ref[...], a3_ref[...],
                    preferred_element_type=jnp.float32)
    t = jnp.tanh(p).reshape(EBLK // 16, 16, HPE)
    out_ref[...] = jnp.sum(t * m2w_ref[...], axis=2) + m2b_ref[...]


@functools.lru_cache(maxsize=None)
def _tc_pre():
    return pl.pallas_call(
        _tc_pre_body,
        out_shape=[jax.ShapeDtypeStruct((NPAD, HP), jnp.float32),
                   jax.ShapeDtypeStruct((NPAD, HID), jnp.float32)],
    )


@functools.lru_cache(maxsize=None)
def _tc_mid():
    return pl.pallas_call(
        _tc_mid_body,
        out_shape=[jax.ShapeDtypeStruct((NPAD, HP), jnp.float32),
                   jax.ShapeDtypeStruct((NPAD, HID), jnp.float32)],
    )


@functools.lru_cache(maxsize=None)
def _tc_post():
    return pl.pallas_call(
        _tc_post_body,
        out_shape=[jax.ShapeDtypeStruct((NPAD, HPE), jnp.bfloat16),
                   jax.ShapeDtypeStruct((NPAD, HPE), jnp.bfloat16),
                   jax.ShapeDtypeStruct((D_EDGE, HPE), jnp.float32)],
    )


@functools.lru_cache(maxsize=None)
def _tc_edge():
    nblk = EPAD // EBLK
    eb16 = EBLK // 16
    nreal = N_EDGES // EBLK
    return pl.pallas_call(
        _tc_edge_body,
        grid=(nblk,),
        in_specs=[
            pl.BlockSpec((EBLK // 8, 8, HPE), lambda i: (i, 0, 0)),
            pl.BlockSpec((EBLK, D_EDGE),
                         lambda i: (jnp.minimum(i, nreal - 1), 0)),
            pl.BlockSpec((D_EDGE, HPE), lambda i: (0, 0)),
            pl.BlockSpec((1, 1, HPE), lambda i: (0, 0, 0)),
            pl.BlockSpec((1, 1), lambda i: (0, 0)),
        ],
        out_specs=pl.BlockSpec((eb16, 16), lambda i: (i, 0)),
        out_shape=jax.ShapeDtypeStruct((EPAD // 16, 16), jnp.float32),
    )


def _pad_cols(a, w=HP):
    return jnp.pad(a, ((0, 0), (0, w - a.shape[1])))


def kernel(x, edge_index, edge_attr, node_w, node_b,
           conv0_init_w, conv0_root_w, conv0_b, bn0_g, bn0_b,
           conv1_init_w, conv1_root_w, conv1_b, bn1_g, bn1_b,
           conv2_init_w, conv2_root_w, conv2_b, bn2_g, bn2_b,
           edge1_w, edge1_b, edge2_w, edge2_b,
           mlp1_w, mlp1_b, mlp2_w, mlp2_b):
    f32 = jnp.float32
    row = edge_index[0]
    col = edge_index[1]
    pad_idx = (N_NODES
               + jnp.arange(EPAD - N_EDGES, dtype=jnp.int32)
               % (NPAD - N_NODES))
    rowp = jnp.concatenate([row.astype(jnp.int32), pad_idx])
    colp = jnp.concatenate([col.astype(jnp.int32), pad_idx])
    xp = jnp.pad(x, ((0, NPAD - N_NODES), (0, 0)))
    zN = jnp.zeros((NPAD,), f32)
    zEB = jnp.zeros((EB, HP), f32)
    row2 = rowp.reshape(EPAD // EB, EB)
    col2 = colp.reshape(EPAD // EB, EB)

    degp = _deg_kernel()(col2, zN)
    degT = degp.T

    r2 = lambda v: v.reshape(1, -1)
    hw0, hr0 = _tc_pre()(xp, node_w, r2(node_b), degT,
                         _pad_cols(conv0_init_w), conv0_root_w, r2(conv0_b))
    aggp0 = _agg_kernel()(hw0, row2, col2, zEB)
    hw1, hr1 = _tc_mid()(aggp0, degT, hr0, r2(bn0_g), r2(bn0_b),
                         _pad_cols(conv1_init_w), conv1_root_w, r2(conv1_b))
    aggp1 = _agg_kernel()(hw1, row2, col2, zEB)
    hw2, hr2 = _tc_mid()(aggp1, degT, hr1, r2(bn1_g), r2(bn1_b),
                         _pad_cols(conv2_init_w), conv2_root_w, r2(conv2_b))
    aggp2 = _agg_kernel()(hw2, row2, col2, zEB)

    m1t_p = _pad_cols(mlp1_w[:HID], HPE)
    m1bh_p = _pad_cols(mlp1_w[HID:], HPE)
    u_tab, w_tab, a3 = _tc_post()(aggp2, degT, hr2, r2(bn2_g), r2(bn2_b),
                                  edge1_w, r2(edge1_b), edge2_w, r2(edge2_b),
                                  m1t_p, m1bh_p, _pad_cols(r2(mlp1_b), HPE))
    id4 = (jnp.arange(4 * EB, dtype=jnp.int32).reshape(4, EB))
    ge = _edge_kernel()(u_tab, w_tab, row2, col2, id4)
    ge = ge.astype(jnp.float32).reshape(EPAD // 8, 8, HPE)
    out = _tc_edge()(ge, edge_attr, a3,
                     _pad_cols(mlp2_w.T, HPE).reshape(1, 1, HPE),
                     mlp2_b.reshape(1, 1))
    return out.reshape(EPAD, 1)[:N_EDGES]


# speedup vs baseline: 1.0839x; 1.0839x over previous
import functools

import jax
import jax.numpy as jnp
from jax import lax
from jax.experimental import pallas as pl
from jax.experimental.pallas import tpu as pltpu
from jax.experimental.pallas import tpu_sc as plsc

N_NODES = 10000
N_EDGES = 320000
D_FEAT = 128
D_EDGE = 16
HID = 72
HP = 80
HPE = 128
NPAD = 10240
EPAD = 327680
NW = 32
EPW = EPAD // NW
EB = 128
NBLK = EPW // EB
RPW = NPAD // 16
NBUF = 4
NGRP = NBLK // NBUF


def _mesh():
    return plsc.VectorSubcoreMesh(core_axis_name="c", subcore_axis_name="s")


_SC_PARAMS = pltpu.CompilerParams(use_tc_tiling_on_sc=False)


@functools.lru_cache(maxsize=None)
def _deg_kernel():
    @functools.partial(
        pl.kernel,
        out_type=jax.ShapeDtypeStruct((2, NPAD), jnp.float32),
        mesh=_mesh(),
        compiler_params=_SC_PARAMS,
        scratch_types=[
            pltpu.VMEM((NBLK, EB), jnp.int32),
            pltpu.VMEM((EB,), jnp.float32),
            pltpu.VMEM_SHARED((NPAD,), jnp.float32),
            pltpu.SemaphoreType.DMA,
        ],
    )
    def deg_k(col2_hbm, zero_hbm, out_hbm, cidx2, ones_v, deg_sp, sem):
        c = lax.axis_index("c")
        s = lax.axis_index("s")
        wid = s * 2 + c
        pltpu.sync_copy(zero_hbm.at[pl.ds(s * RPW, RPW)],
                        deg_sp.at[pl.ds(s * RPW, RPW)])
        pltpu.sync_copy(col2_hbm.at[pl.ds(wid * NBLK, NBLK)], cidx2)
        for j in range(EB // 16):
            ones_v[pl.ds(j * 16, 16)] = jnp.ones((16,), jnp.float32)
        plsc.subcore_barrier()

        def grp(G, carry):
            g0 = G * NBUF
            descs = [
                pltpu.async_copy(ones_v, deg_sp.at[cidx2.at[g0 + b]], sem,
                                 add=True)
                for b in range(NBUF)
            ]
            for d in descs:
                d.wait()
            return carry

        lax.fori_loop(0, NGRP, grp, 0)
        plsc.subcore_barrier()
        pltpu.sync_copy(deg_sp.at[pl.ds(s * RPW, RPW)],
                        out_hbm.at[c, pl.ds(s * RPW, RPW)])

    return deg_k


@functools.lru_cache(maxsize=None)
def _agg_kernel():
    @functools.partial(
        pl.kernel,
        out_type=jax.ShapeDtypeStruct((2, NPAD, HP), jnp.float32),
        mesh=_mesh(),
        compiler_params=_SC_PARAMS,
        scratch_types=[
            pltpu.VMEM((NBLK, EB), jnp.int32),
            pltpu.VMEM((NBLK, EB), jnp.int32),
            pltpu.VMEM((NBUF * EB, HP), jnp.float32),
            pltpu.VMEM_SHARED((NPAD, HP), jnp.float32),
            pltpu.SemaphoreType.DMA,
            pltpu.SemaphoreType.DMA,
        ],
    )
    def agg_k(tab_hbm, row2_hbm, col2_hbm, zero_hbm, out_hbm,
              ridx2, cidx2, gball, agg_sp, gsem, ssem):
        c = lax.axis_index("c")
        s = lax.axis_index("s")
        wid = s * 2 + c
        for j in range(RPW // EB):
            pltpu.sync_copy(zero_hbm,
                            agg_sp.at[pl.ds(s * RPW + j * EB, EB)])
        pltpu.sync_copy(row2_hbm.at[pl.ds(wid * NBLK, NBLK)], ridx2)
        pltpu.sync_copy(col2_hbm.at[pl.ds(wid * NBLK, NBLK)], cidx2)
        plsc.subcore_barrier()

        NB2 = NBUF // 2
        NG2 = NBLK // NB2

        def gath(G, half):
            return [
                pltpu.async_copy(
                    tab_hbm.at[ridx2.at[G * NB2 + b]],
                    gball.at[pl.ds((half * NB2 + b) * EB, EB)], gsem)
                for b in range(NB2)
            ]

        def scat(G, half):
            return [
                pltpu.async_copy(
                    gball.at[pl.ds((half * NB2 + b) * EB, EB)],
                    agg_sp.at[cidx2.at[G * NB2 + b]], ssem, add=True)
                for b in range(NB2)
            ]

        gds = gath(0, 0)
        sds_prev = None
        sds_tail = []
        for G in range(NG2):
            h = G % 2
            for d in gds:
                d.wait()
            sds = scat(G, h)
            if G + 1 < NG2:
                if sds_prev is not None:
                    for d in sds_prev:
                        d.wait()
                gds = gath(G + 1, 1 - h)
            else:
                sds_tail = sds_prev or []
            sds_prev = sds
        for d in sds_tail + sds_prev:
            d.wait()
        plsc.subcore_barrier()
        pltpu.sync_copy(agg_sp.at[pl.ds(s * RPW, RPW)],
                        out_hbm.at[c, pl.ds(s * RPW, RPW)])

    return agg_k


@functools.lru_cache(maxsize=None)
def _edge_kernel():
    @functools.partial(
        pl.kernel,
        out_type=jax.ShapeDtypeStruct((EPAD, HPE), jnp.bfloat16),
        mesh=_mesh(),
        compiler_params=_SC_PARAMS,
        scratch_types=[
            pltpu.VMEM((NBLK, EB), jnp.int32),
            pltpu.VMEM((NBLK, EB), jnp.int32),
            pltpu.VMEM((4, EB), jnp.int32),
            pltpu.VMEM((4 * EB, HPE), jnp.bfloat16),
            pltpu.VMEM((4 * EB, HPE), jnp.bfloat16),
            pltpu.VMEM_SHARED((16 * 4 * EB, HPE), jnp.bfloat16),
            pltpu.SemaphoreType.DMA,
            pltpu.SemaphoreType.DMA,
            pltpu.SemaphoreType.DMA,
        ],
    )
    def edge_k(u_hbm, w_hbm, row2_hbm, col2_hbm, id4_hbm, out_hbm,
               ridx2, cidx2, iref, uball, wball, zone, gsem, asem, wsem):
        c = lax.axis_index("c")
        s = lax.axis_index("s")
        wid = s * 2 + c
        pltpu.sync_copy(row2_hbm.at[pl.ds(wid * NBLK, NBLK)], ridx2)
        pltpu.sync_copy(col2_hbm.at[pl.ds(wid * NBLK, NBLK)], cidx2)
        pltpu.sync_copy(id4_hbm, iref)
        for r in range(4):
            for j in range(EB // 16):
                sl = pl.ds(j * 16, 16)
                iref[r, sl] = iref[r, sl] + s * (4 * EB)
        base = wid * EPW

        NB2 = 2
        NG2 = NBLK // NB2

        def zsl(half, b):
            return pl.ds((s * 4 + half * NB2 + b) * EB, EB)

        def gath(G, half):
            gds = []
            for b in range(NB2):
                sl = pl.ds((half * NB2 + b) * EB, EB)
                gds.append(pltpu.async_copy(
                    u_hbm.at[ridx2.at[G * NB2 + b]], uball.at[sl], gsem))
                gds.append(pltpu.async_copy(
                    w_hbm.at[cidx2.at[G * NB2 + b]], wball.at[sl], gsem))
            return gds

        def setu(half):
            return [
                pltpu.async_copy(
                    uball.at[pl.ds((half * NB2 + b) * EB, EB)],
                    zone.at[zsl(half, b)], asem)
                for b in range(NB2)
            ]

        def addw(half):
            return [
                pltpu.async_copy(
                    wball.at[pl.ds((half * NB2 + b) * EB, EB)],
                    zone.at[iref.at[half * NB2 + b]], asem, add=True)
                for b in range(NB2)
            ]

        def wrt(G, half):
            return [
                pltpu.async_copy(
                    zone.at[zsl(half, b)],
                    out_hbm.at[pl.ds(base + (G * NB2 + b) * EB, EB)], wsem)
                for b in range(NB2)
            ]

        gds = gath(0, 0)
        wds_prev = None
        wds_tail = []
        for G in range(NG2):
            h = G % 2
            for d in gds:
                d.wait()
            for d in setu(h):
                d.wait()
            for d in addw(h):
                d.wait()
            wds = wrt(G, h)
            if G + 1 < NG2:
                if wds_prev is not None:
                    for d in wds_prev:
                        d.wait()
                gds = gath(G + 1, 1 - h)
            else:
                wds_tail = wds_prev or []
            wds_prev = wds
        for d in wds_tail + wds_prev:
            d.wait()

    return edge_k


def _dis_from_degT(degT):
    deg = degT[:, 0:1] + degT[:, 1:2]
    return jnp.where(deg > 0.0, lax.rsqrt(jnp.maximum(deg, 1e-12)), 0.0)


def _tc_pre_body(x_ref, nw_ref, nb_ref, degT_ref, iw_ref, rw_ref, cb_ref,
                 hw_ref, hr_ref):
    dis = _dis_from_degT(degT_ref[...])
    h0 = jnp.maximum(
        jnp.dot(x_ref[...], nw_ref[...], preferred_element_type=jnp.float32)
        + nb_ref[...], 0.0)
    hw_ref[...] = dis * jnp.dot(h0, iw_ref[...],
                                preferred_element_type=jnp.float32)
    hr_ref[...] = jnp.dot(h0, rw_ref[...],
                          preferred_element_type=jnp.float32) + cb_ref[...]


def _bn_layer(aggp, degT, hr, g, b):
    dis = _dis_from_degT(degT)
    agg = (aggp[0] + aggp[1])[:, :HID] * dis
    h2 = jnp.maximum(agg + hr, 0.0)
    mask = (lax.broadcasted_iota(jnp.int32, (NPAD, 1), 0)
            < N_NODES).astype(jnp.float32)
    h2m = h2 * mask
    mu = jnp.sum(h2m, axis=0, keepdims=True) / N_NODES
    var = jnp.sum((h2 - mu) ** 2 * mask, axis=0, keepdims=True) / N_NODES
    hn = (h2 - mu) * lax.rsqrt(var + 1e-5) * g + b
    return hn, dis


def _tc_mid_body(aggp_ref, degT_ref, hr_ref, g_ref, b_ref,
                 iw_ref, rw_ref, cb_ref, hw_ref, hrn_ref):
    hn, dis = _bn_layer(aggp_ref[...], degT_ref[...], hr_ref[...],
                        g_ref[...], b_ref[...])
    hw_ref[...] = dis * jnp.dot(hn, iw_ref[...],
                                preferred_element_type=jnp.float32)
    hrn_ref[...] = jnp.dot(hn, rw_ref[...],
                           preferred_element_type=jnp.float32) + cb_ref[...]


def _tc_post_body(aggp_ref, degT_ref, hr_ref, g_ref, b_ref,
                  e1w_ref, e1b_ref, e2w_ref, e2b_ref,
                  m1t_ref, m1bh_ref, m1b_ref,
                  u_ref, w_ref, a3_ref):
    hn, _ = _bn_layer(aggp_ref[...], degT_ref[...], hr_ref[...],
                      g_ref[...], b_ref[...])
    e1t = e1w_ref[0:HID, :]
    e1bm = e1w_ref[HID:2 * HID, :]
    m1t = m1t_ref[...]
    m1bh = m1bh_ref[...]
    bias = (jnp.dot(e1b_ref[...], m1t, preferred_element_type=jnp.float32)
            + jnp.dot(e2b_ref[...], m1bh, preferred_element_type=jnp.float32)
            + m1b_ref[...])
    u_ref[...] = (jnp.dot(
        jnp.dot(hn, e1t, preferred_element_type=jnp.float32), m1t,
        preferred_element_type=jnp.float32) + bias).astype(jnp.bfloat16)
    w_ref[...] = jnp.dot(
        jnp.dot(hn, e1bm, preferred_element_type=jnp.float32), m1t,
        preferred_element_type=jnp.float32).astype(jnp.bfloat16)
    a3_ref[...] = jnp.dot(e2w_ref[...], m1bh,
                          preferred_element_type=jnp.float32)


EBLK = 2560


def _tc_edge_body(g_ref, attr_ref, a3_ref, m2w_ref, m2b_ref, out_ref):
    p = (g_ref[...].astype(jnp.float32)
         + jnp.dot(attr_ref[...], a3_ref[...],
                   preferred_element_type=jnp.float32))
    t = jnp.tanh(p).reshape(EBLK // 16, 16, HPE)
    out_ref[...] = jnp.sum(t * m2w_ref[...], axis=2) + m2b_ref[...]


@functools.lru_cache(maxsize=None)
def _tc_pre():
    return pl.pallas_call(
        _tc_pre_body,
        out_shape=[jax.ShapeDtypeStruct((NPAD, HP), jnp.float32),
                   jax.ShapeDtypeStruct((NPAD, HID), jnp.float32)],
    )


@functools.lru_cache(maxsize=None)
def _tc_mid():
    return pl.pallas_call(
        _tc_mid_body,
        out_shape=[jax.ShapeDtypeStruct((NPAD, HP), jnp.float32),
                   jax.ShapeDtypeStruct((NPAD, HID), jnp.float32)],
    )


@functools.lru_cache(maxsize=None)
def _tc_post():
    return pl.pallas_call(
        _tc_post_body,
        out_shape=[jax.ShapeDtypeStruct((NPAD, HPE), jnp.bfloat16),
                   jax.ShapeDtypeStruct((NPAD, HPE), jnp.bfloat16),
                   jax.ShapeDtypeStruct((D_EDGE, HPE), jnp.float32)],
    )


@functools.lru_cache(maxsize=None)
def _tc_edge():
    nblk = EPAD // EBLK
    eb16 = EBLK // 16
    nreal = N_EDGES // EBLK
    return pl.pallas_call(
        _tc_edge_body,
        grid=(nblk,),
        in_specs=[
            pl.BlockSpec((EBLK, HPE), lambda i: (i, 0)),
            pl.BlockSpec((EBLK, D_EDGE),
                         lambda i: (jnp.minimum(i, nreal - 1), 0)),
            pl.BlockSpec((D_EDGE, HPE), lambda i: (0, 0)),
            pl.BlockSpec((1, 1, HPE), lambda i: (0, 0, 0)),
            pl.BlockSpec((1, 1), lambda i: (0, 0)),
        ],
        out_specs=pl.BlockSpec((eb16, 16), lambda i: (i, 0)),
        out_shape=jax.ShapeDtypeStruct((EPAD // 16, 16), jnp.float32),
    )


def _pad_cols(a, w=HP):
    return jnp.pad(a, ((0, 0), (0, w - a.shape[1])))


def kernel(x, edge_index, edge_attr, node_w, node_b,
           conv0_init_w, conv0_root_w, conv0_b, bn0_g, bn0_b,
           conv1_init_w, conv1_root_w, conv1_b, bn1_g, bn1_b,
           conv2_init_w, conv2_root_w, conv2_b, bn2_g, bn2_b,
           edge1_w, edge1_b, edge2_w, edge2_b,
           mlp1_w, mlp1_b, mlp2_w, mlp2_b):
    f32 = jnp.float32
    row = edge_index[0]
    col = edge_index[1]
    pad_idx = (N_NODES
               + jnp.arange(EPAD - N_EDGES, dtype=jnp.int32)
               % (NPAD - N_NODES))
    rowp = jnp.concatenate([row.astype(jnp.int32), pad_idx])
    colp = jnp.concatenate([col.astype(jnp.int32), pad_idx])
    xp = jnp.pad(x, ((0, NPAD - N_NODES), (0, 0)))
    zN = jnp.zeros((NPAD,), f32)
    zEB = jnp.zeros((EB, HP), f32)
    row2 = rowp.reshape(EPAD // EB, EB)
    col2 = colp.reshape(EPAD // EB, EB)

    degp = _deg_kernel()(col2, zN)
    degT = degp.T

    r2 = lambda v: v.reshape(1, -1)
    hw0, hr0 = _tc_pre()(xp, node_w, r2(node_b), degT,
                         _pad_cols(conv0_init_w), conv0_root_w, r2(conv0_b))
    aggp0 = _agg_kernel()(hw0, row2, col2, zEB)
    hw1, hr1 = _tc_mid()(aggp0, degT, hr0, r2(bn0_g), r2(bn0_b),
                         _pad_cols(conv1_init_w), conv1_root_w, r2(conv1_b))
    aggp1 = _agg_kernel()(hw1, row2, col2, zEB)
    hw2, hr2 = _tc_mid()(aggp1, degT, hr1, r2(bn1_g), r2(bn1_b),
                         _pad_cols(conv2_init_w), conv2_root_w, r2(conv2_b))
    aggp2 = _agg_kernel()(hw2, row2, col2, zEB)

    m1t_p = _pad_cols(mlp1_w[:HID], HPE)
    m1bh_p = _pad_cols(mlp1_w[HID:], HPE)
    u_tab, w_tab, a3 = _tc_post()(aggp2, degT, hr2, r2(bn2_g), r2(bn2_b),
                                  edge1_w, r2(edge1_b), edge2_w, r2(edge2_b),
                                  m1t_p, m1bh_p, _pad_cols(r2(mlp1_b), HPE))
    id4 = (jnp.arange(4 * EB, dtype=jnp.int32).reshape(4, EB))
    ge = _edge_kernel()(u_tab, w_tab, row2, col2, id4)
    out = _tc_edge()(ge, edge_attr, a3,
                     _pad_cols(mlp2_w.T, HPE).reshape(1, 1, HPE),
                     mlp2_b.reshape(1, 1))
    return out.reshape(EPAD, 1)[:N_EDGES]
